# baseline (speedup 1.0000x reference)
import jax
import jax.numpy as jnp
from jax import lax
from jax.experimental import pallas as pl
from jax.experimental.pallas import tpu as pltpu
from jax.experimental.pallas import tpu_sc as plsc

D_MODEL = 64
BATCH = 4096
SEQ_LEN = 200
B_TOTAL = BATCH * SEQ_LEN
NUM_CORES = 2
NUM_SUBCORES = 16
NW = NUM_CORES * NUM_SUBCORES
B_PER_W = B_TOTAL // NW
CHUNK = 400
N_CHUNK = B_PER_W // CHUNK
NBUF = 4


def _emb_body(x_hbm, table_hbm, out_hbm, idx_v, rbufs, gsems, wsems):
    c = lax.axis_index("c")
    s = lax.axis_index("s")
    wid = s * NUM_CORES + c
    base = wid * B_PER_W

    def idx_slice(g):
        return idx_v.at[pl.ds(g * CHUNK, CHUNK)]

    def start_gather(g, b):
        pltpu.async_copy(table_hbm.at[idx_slice(g)], rbufs[b], gsems[b])

    def wait_gather(b):
        pltpu.make_async_copy(table_hbm.at[idx_slice(0)], rbufs[b],
                              gsems[b]).wait()

    def start_wb(g, b):
        pltpu.async_copy(rbufs[b], out_hbm.at[pl.ds(base + g * CHUNK, CHUNK)],
                         wsems[b])

    def wait_wb(b):
        pltpu.make_async_copy(rbufs[b], out_hbm.at[pl.ds(base, CHUNK)],
                              wsems[b]).wait()

    pltpu.sync_copy(x_hbm.at[pl.ds(base, B_PER_W)], idx_v)

    start_gather(0, 0)
    start_gather(1, 1)
    for j in range(NBUF):
        if j + 2 < N_CHUNK:
            if j >= 2:
                wait_wb((j + 2) % NBUF)
            start_gather(j + 2, (j + 2) % NBUF)
        wait_gather(j)
        start_wb(j, j)

    def body(t, carry):
        g0 = t * NBUF
        for j in range(NBUF):
            b = j
            b2 = (j + 2) % NBUF
            wait_wb(b2)
            start_gather(g0 + j + 2, b2)
            wait_gather(b)
            start_wb(g0 + j, b)
        return carry

    lax.fori_loop(1, N_CHUNK // NBUF - 1, body, 0)

    for j in range(NBUF):
        g = N_CHUNK - NBUF + j
        b = g % NBUF
        if j < 2:
            b2 = (g + 2) % NBUF
            wait_wb(b2)
            start_gather(g + 2, b2)
        wait_gather(b)
        start_wb(g, b)
    for b in range(NBUF):
        wait_wb(b)


@jax.jit
def kernel(x, table):
    xf = x.reshape(B_TOTAL)
    out = pl.kernel(
        _emb_body,
        out_type=jax.ShapeDtypeStruct((B_TOTAL, D_MODEL), jnp.float32),
        mesh=plsc.VectorSubcoreMesh(core_axis_name="c", subcore_axis_name="s"),
        compiler_params=pltpu.CompilerParams(use_tc_tiling_on_sc=False),
        scratch_types=[
            pltpu.VMEM((B_PER_W,), jnp.int32),
            [pltpu.VMEM((CHUNK, D_MODEL), jnp.float32) for _ in range(NBUF)],
            [pltpu.SemaphoreType.DMA for _ in range(NBUF)],
            [pltpu.SemaphoreType.DMA for _ in range(NBUF)],
        ],
    )(xf, table)
    return out.reshape(BATCH, SEQ_LEN, D_MODEL)
